# baseline (speedup 1.0000x reference)
import functools

import jax
import jax.numpy as jnp
from jax import lax
from jax.experimental import pallas as pl
from jax.experimental.pallas import tpu as pltpu
from jax.experimental.pallas import tpu_sc as plsc

N_TOKENS = 3200000
TEXT_LEN = 100000
TP = 100096
GP = 8
VOCAB = 64
L = 4

NC = 2
NS = 16
NW = NC * NS
TPW = N_TOKENS // NW
STRIPE = TP // NS

C1 = 2000
NCH1 = TPW // C1
C3 = 4000
NCH3 = TPW // C3
CH = 512
HB = CH + 48

_mesh = plsc.VectorSubcoreMesh(core_axis_name="c", subcore_axis_name="s")


def _k1_body(aug_hbm, sp_hbm, tid_hbm, zeros_hbm, out_hbm,
             aug_sh, s_sh, pos_a, pos_b, pos_c, pos_d, tid_a, tid_b,
             vals_a, vals_b, sem_p, sem_t, sem_g, sem_s0, sem_s1):
    c = lax.axis_index("c")
    s = lax.axis_index("s")
    wid = s * NC + c
    base = wid * TPW
    r0 = s * STRIPE

    pltpu.sync_copy(zeros_hbm, s_sh.at[pl.ds(r0, STRIPE)])

    @pl.when(s == 0)
    def _():
        pltpu.sync_copy(aug_hbm, aug_sh)
        pltpu.sync_copy(zeros_hbm.at[pl.ds(0, GP)], out_hbm.at[c, pl.ds(0, GP)])
        pltpu.sync_copy(zeros_hbm.at[pl.ds(0, GP)],
                        out_hbm.at[c, pl.ds(GP + TP, GP)])

    plsc.subcore_barrier()

    pos_bufs = [pos_a, pos_b, pos_c, pos_d]
    tid_bufs = [tid_a, tid_b]
    vals_bufs = [vals_a, vals_b]
    pend = [(pltpu.async_copy(sp_hbm.at[pl.ds(base, C1)], pos_a, sem_p),
             pltpu.async_copy(tid_hbm.at[pl.ds(base, C1)], tid_a, sem_t))]
    sc_pend = []
    for j in range(NCH1):
        cpj, ctj = pend.pop()
        cpj.wait()
        ctj.wait()
        if j + 1 < NCH1:
            nb = base + (j + 1) * C1
            pend.append(
                (pltpu.async_copy(sp_hbm.at[pl.ds(nb, C1)],
                                  pos_bufs[(j + 1) % 4], sem_p),
                 pltpu.async_copy(tid_hbm.at[pl.ds(nb, C1)],
                                  tid_bufs[(j + 1) % 2], sem_t)))
        if len(sc_pend) >= 2:
            sc_pend.pop(0).wait()
        vb = vals_bufs[j % 2]
        pltpu.async_copy(aug_sh.at[tid_bufs[j % 2]], vb, sem_g).wait()
        sc_pend.append(
            pltpu.async_copy(vb, s_sh.at[pos_bufs[j % 4]],
                             [sem_s0, sem_s1][j % 2], add=True))
    for d in sc_pend:
        d.wait()

    plsc.subcore_barrier()
    pltpu.sync_copy(s_sh.at[pl.ds(r0, STRIPE)],
                    out_hbm.at[c, pl.ds(GP + r0, STRIPE)])


_k1 = functools.partial(
    pl.kernel,
    out_type=jax.ShapeDtypeStruct((NC, TP + 2 * GP, 8), jnp.float32),
    mesh=_mesh,
    compiler_params=pltpu.CompilerParams(use_tc_tiling_on_sc=False,
                                         disable_bounds_checks=True),
    scratch_types=[
        pltpu.VMEM_SHARED((VOCAB, 8), jnp.float32),
        pltpu.VMEM_SHARED((TP, 8), jnp.float32),
        pltpu.VMEM((C1,), jnp.int32),
        pltpu.VMEM((C1,), jnp.int32),
        pltpu.VMEM((C1,), jnp.int32),
        pltpu.VMEM((C1,), jnp.int32),
        pltpu.VMEM((C1,), jnp.int32),
        pltpu.VMEM((C1,), jnp.int32),
        pltpu.VMEM((C1, 8), jnp.float32),
        pltpu.VMEM((C1, 8), jnp.float32),
        pltpu.SemaphoreType.DMA,
        pltpu.SemaphoreType.DMA,
        pltpu.SemaphoreType.DMA,
        pltpu.SemaphoreType.DMA,
        pltpu.SemaphoreType.DMA,
    ],
)(_k1_body)


def _chunks():
    out = []
    off = 0
    while off < STRIPE:
        ch = min(CH, STRIPE - off)
        out.append((off, ch))
        off += ch
    return out


def _k23_body(part_hbm, sp_hbm, out_hbm,
              w4_sh, h0, h1, bw_v, w4c_v, w4_v, idx_a, idx_b, out_v,
              sem_h, sem_i):
    c = lax.axis_index("c")
    s = lax.axis_index("s")
    wid = s * NC + c
    base = wid * TPW
    r0 = s * STRIPE

    iota = lax.iota(jnp.int32, 16)
    c4 = jnp.full((16,), 4, jnp.int32)

    for off, ch in _chunks():
        a = r0 + off
        nrow = ch + 2 * GP
        cp0 = pltpu.async_copy(part_hbm.at[0, pl.ds(a, nrow)],
                               h0.at[pl.ds(0, nrow)], sem_h)
        cp1 = pltpu.async_copy(part_hbm.at[1, pl.ds(a, nrow)],
                               h1.at[pl.ds(0, nrow)], sem_h)
        cp0.wait()
        cp1.wait()

        ng1 = (ch + 3 + 15) // 16

        @plsc.parallel_loop(0, ng1, unroll=4)
        def _(g):
            l = pl.multiple_of(g * 16, 16) + GP
            rq = jnp.full((16,), l, jnp.int32) + iota
            bsum = jnp.zeros((16,), jnp.float32)
            bcnt = jnp.zeros((16,), jnp.float32)
            for k in range(4):
                rk = rq - k
                ck = jnp.full((16,), k, jnp.int32)
                bsum = bsum + plsc.load_gather(h0, [rk, ck])
                bsum = bsum + plsc.load_gather(h1, [rk, ck])
                bcnt = bcnt + plsc.load_gather(h0, [rk, c4])
                bcnt = bcnt + plsc.load_gather(h1, [rk, c4])
            bw = jnp.where(bcnt > 0, bsum / jnp.maximum(bcnt, 1.0), 0.0)
            bw_v[pl.ds(l, 16)] = bw

        ng2 = (ch + 15) // 16

        @plsc.parallel_loop(0, ng2, unroll=4)
        def _(g):
            o = pl.multiple_of(g * 16, 16)
            lp = o + GP
            rq = jnp.full((16,), lp, jnp.int32) + iota
            w = bw_v[pl.ds(lp, 16)]
            for k in range(1, 4):
                w = w + plsc.load_gather(bw_v, [rq + k])
            w4c_v[pl.ds(o, 16)] = w

        pltpu.sync_copy(w4c_v.at[pl.ds(0, ch)], w4_sh.at[pl.ds(a, ch)])

    plsc.subcore_barrier()

    pltpu.sync_copy(w4_sh, w4_v)

    idx_bufs = [idx_a, idx_b]
    pend = [pltpu.async_copy(sp_hbm.at[pl.ds(base, C3)], idx_a, sem_i)]
    for j in range(NCH3):
        pend.pop().wait()
        if j + 1 < NCH3:
            pend.append(pltpu.async_copy(
                sp_hbm.at[pl.ds(base + (j + 1) * C3, C3)],
                idx_bufs[(j + 1) % 2], sem_i))
        ib = idx_bufs[j % 2]

        @plsc.parallel_loop(0, C3 // 16, unroll=8)
        def _(g):
            off = pl.multiple_of(g * 16, 16)
            idx = ib[pl.ds(off, 16)]
            out_v[pl.ds(off, 16)] = plsc.load_gather(w4_v, [idx])

        pltpu.sync_copy(out_v, out_hbm.at[pl.ds(base + j * C3, C3)])


_k23 = functools.partial(
    pl.kernel,
    out_type=jax.ShapeDtypeStruct((N_TOKENS,), jnp.float32),
    mesh=_mesh,
    compiler_params=pltpu.CompilerParams(use_tc_tiling_on_sc=False,
                                         needs_layout_passes=False,
                                         disable_bounds_checks=True),
    scratch_types=[
        pltpu.VMEM_SHARED((TP,), jnp.float32),
        pltpu.VMEM((HB, 8), jnp.float32),
        pltpu.VMEM((HB, 8), jnp.float32),
        pltpu.VMEM((HB,), jnp.float32),
        pltpu.VMEM((CH,), jnp.float32),
        pltpu.VMEM((TP,), jnp.float32),
        pltpu.VMEM((C3,), jnp.int32),
        pltpu.VMEM((C3,), jnp.int32),
        pltpu.VMEM((C3,), jnp.float32),
        pltpu.SemaphoreType.DMA,
        pltpu.SemaphoreType.DMA,
    ],
)(_k23_body)


def kernel(byte_params, start_pos, token_id, text_len):
    bp = byte_params.astype(jnp.float32)
    aug = jnp.concatenate(
        [bp, jnp.ones((VOCAB, 1), jnp.float32),
         jnp.zeros((VOCAB, 3), jnp.float32)], axis=1)
    zeros_stripe = jnp.zeros((STRIPE, 8), jnp.float32)
    partial = _k1(aug, start_pos, token_id, zeros_stripe)
    return _k23(partial, start_pos)


# speedup vs baseline: 1.0959x; 1.0959x over previous
import functools

import jax
import jax.numpy as jnp
from jax import lax
from jax.experimental import pallas as pl
from jax.experimental.pallas import tpu as pltpu
from jax.experimental.pallas import tpu_sc as plsc

N_TOKENS = 3200000
TEXT_LEN = 100000
TP = 100096
GP = 8
VOCAB = 64
L = 4

NC = 2
NS = 16
NW = NC * NS
TPW = N_TOKENS // NW
STRIPE = TP // NS

C1 = 2000
NCH1 = TPW // C1
C3 = 2000
NCH3 = TPW // C3
CH = 368
NCHD = STRIPE // CH
NR = CH + 2 * GP
HB = CH + 48
_mesh = plsc.VectorSubcoreMesh(core_axis_name="c", subcore_axis_name="s")


def _k1_body(aug_hbm, sp_hbm, tid_hbm, zeros_hbm, out_hbm,
             aug_sh, s_sh, pos_a, pos_b, pos_c, pos_d, tid_a, tid_b,
             vals_a, vals_b, sem_p, sem_t, sem_g, sem_s0, sem_s1):
    c = lax.axis_index("c")
    s = lax.axis_index("s")
    wid = s * NC + c
    base = wid * TPW
    r0 = s * STRIPE

    pltpu.sync_copy(zeros_hbm, s_sh.at[pl.ds(r0, STRIPE)])

    @pl.when(s == 0)
    def _():
        pltpu.sync_copy(aug_hbm, aug_sh)
        pltpu.sync_copy(zeros_hbm.at[pl.ds(0, GP)], out_hbm.at[c, pl.ds(0, GP)])
        pltpu.sync_copy(zeros_hbm.at[pl.ds(0, GP)],
                        out_hbm.at[c, pl.ds(GP + TP, GP)])

    plsc.subcore_barrier()

    pos_bufs = [pos_a, pos_b, pos_c, pos_d]
    tid_bufs = [tid_a, tid_b]
    vals_bufs = [vals_a, vals_b]
    pend = [(pltpu.async_copy(sp_hbm.at[pl.ds(base, C1)], pos_a, sem_p),
             pltpu.async_copy(tid_hbm.at[pl.ds(base, C1)], tid_a, sem_t))]
    sc_pend = []
    for j in range(NCH1):
        cpj, ctj = pend.pop()
        cpj.wait()
        ctj.wait()
        if j + 1 < NCH1:
            nb = base + (j + 1) * C1
            pend.append(
                (pltpu.async_copy(sp_hbm.at[pl.ds(nb, C1)],
                                  pos_bufs[(j + 1) % 4], sem_p),
                 pltpu.async_copy(tid_hbm.at[pl.ds(nb, C1)],
                                  tid_bufs[(j + 1) % 2], sem_t)))
        if len(sc_pend) >= 2:
            sc_pend.pop(0).wait()
        vb = vals_bufs[j % 2]
        pltpu.async_copy(aug_sh.at[tid_bufs[j % 2]], vb, sem_g).wait()
        sc_pend.append(
            pltpu.async_copy(vb, s_sh.at[pos_bufs[j % 4]],
                             [sem_s0, sem_s1][j % 2], add=True))
    for d in sc_pend:
        d.wait()

    plsc.subcore_barrier()
    pltpu.sync_copy(s_sh.at[pl.ds(r0, STRIPE)],
                    out_hbm.at[c, pl.ds(GP + r0, STRIPE)])


_k1 = functools.partial(
    pl.kernel,
    out_type=jax.ShapeDtypeStruct((NC, TP + 2 * GP, 8), jnp.float32),
    mesh=_mesh,
    compiler_params=pltpu.CompilerParams(use_tc_tiling_on_sc=False,
                                         disable_bounds_checks=True),
    scratch_types=[
        pltpu.VMEM_SHARED((VOCAB, 8), jnp.float32),
        pltpu.VMEM_SHARED((TP, 8), jnp.float32),
        pltpu.VMEM((C1,), jnp.int32),
        pltpu.VMEM((C1,), jnp.int32),
        pltpu.VMEM((C1,), jnp.int32),
        pltpu.VMEM((C1,), jnp.int32),
        pltpu.VMEM((C1,), jnp.int32),
        pltpu.VMEM((C1,), jnp.int32),
        pltpu.VMEM((C1, 8), jnp.float32),
        pltpu.VMEM((C1, 8), jnp.float32),
        pltpu.SemaphoreType.DMA,
        pltpu.SemaphoreType.DMA,
        pltpu.SemaphoreType.DMA,
        pltpu.SemaphoreType.DMA,
        pltpu.SemaphoreType.DMA,
    ],
)(_k1_body)


def _k23_body(part_hbm, sp_hbm, out_hbm,
              w4_sh, h0a, h1a, h0b, h1b, bw_v, w4c_v, w4_v,
              idx_a, idx_b, out_a, out_b,
              sem_ha, sem_hb, sem_ia, sem_ib, sem_oa, sem_ob):
    c = lax.axis_index("c")
    s = lax.axis_index("s")
    wid = s * NC + c
    base = wid * TPW
    r0 = s * STRIPE

    iota = lax.iota(jnp.int32, 16)
    c4 = jnp.full((16,), 4, jnp.int32)

    hb_bufs = [(h0a, h1a, sem_ha), (h0b, h1b, sem_hb)]

    def _halo_issue(j, b):
        a = r0 + j * CH
        b0, b1, sm = hb_bufs[b]
        pltpu.async_copy(part_hbm.at[0, pl.ds(a, NR)], b0.at[pl.ds(0, NR)], sm)
        pltpu.async_copy(part_hbm.at[1, pl.ds(a, NR)], b1.at[pl.ds(0, NR)], sm)

    def _halo_wait(b):
        b0, b1, sm = hb_bufs[b]
        pltpu.make_async_copy(part_hbm.at[0, pl.ds(0, NR)],
                              b0.at[pl.ds(0, NR)], sm).wait()
        pltpu.make_async_copy(part_hbm.at[1, pl.ds(0, NR)],
                              b1.at[pl.ds(0, NR)], sm).wait()

    def _dense_chunk(j, b):
        a = r0 + j * CH
        _halo_wait(b)
        h0, h1, _ = hb_bufs[b]

        ng1 = (CH + 3 + 15) // 16

        @plsc.parallel_loop(0, ng1, unroll=4)
        def _(g):
            l = pl.multiple_of(g * 16, 16) + GP
            rq = jnp.full((16,), l, jnp.int32) + iota
            bsum = jnp.zeros((16,), jnp.float32)
            bcnt = jnp.zeros((16,), jnp.float32)
            for k in range(4):
                rk = rq - k
                ck = jnp.full((16,), k, jnp.int32)
                bsum = bsum + plsc.load_gather(h0, [rk, ck])
                bsum = bsum + plsc.load_gather(h1, [rk, ck])
                bcnt = bcnt + plsc.load_gather(h0, [rk, c4])
                bcnt = bcnt + plsc.load_gather(h1, [rk, c4])
            bw = jnp.where(bcnt > 0, bsum / jnp.maximum(bcnt, 1.0), 0.0)
            bw_v[pl.ds(l, 16)] = bw

        @plsc.parallel_loop(0, CH // 16, unroll=4)
        def _(g):
            o = pl.multiple_of(g * 16, 16)
            rq = jnp.full((16,), o + GP, jnp.int32) + iota
            w = bw_v[pl.ds(o + GP, 16)]
            for k in range(1, 4):
                w = w + plsc.load_gather(bw_v, [rq + k])
            w4c_v[pl.ds(o, 16)] = w

        pltpu.sync_copy(w4c_v, w4_sh.at[pl.ds(a, CH)])

        @pl.when(j + 2 < NCHD)
        def _():
            _halo_issue(j + 2, b)

    _halo_issue(0, 0)
    _halo_issue(1, 1)

    @pl.loop(0, NCHD // 2)
    def _(i):
        _dense_chunk(2 * i, 0)
        _dense_chunk(2 * i + 1, 1)

    _dense_chunk(NCHD - 1, 0)

    plsc.subcore_barrier()

    pltpu.sync_copy(w4_sh, w4_v)

    idx_bufs = [(idx_a, sem_ia), (idx_b, sem_ib)]
    out_bufs = [(out_a, sem_oa), (out_b, sem_ob)]

    def _idx_issue(j, b):
        ib, sm = idx_bufs[b]
        pltpu.async_copy(sp_hbm.at[pl.ds(base + j * C3, C3)], ib, sm)

    def _gather_chunk(j, b):
        ib, smi = idx_bufs[b]
        ob, smo = out_bufs[b]
        pltpu.make_async_copy(sp_hbm.at[pl.ds(base, C3)], ib, smi).wait()

        @pl.when(j >= 2)
        def _():
            pltpu.make_async_copy(ob, out_hbm.at[pl.ds(base, C3)], smo).wait()

        @plsc.parallel_loop(0, C3 // 16, unroll=8)
        def _(g):
            off = pl.multiple_of(g * 16, 16)
            idx = ib[pl.ds(off, 16)]
            ob[pl.ds(off, 16)] = plsc.load_gather(w4_v, [idx])

        pltpu.async_copy(ob, out_hbm.at[pl.ds(base + j * C3, C3)], smo)

        @pl.when(j + 2 < NCH3)
        def _():
            _idx_issue(j + 2, b)

    _idx_issue(0, 0)
    _idx_issue(1, 1)

    @pl.loop(0, NCH3 // 2)
    def _(i):
        _gather_chunk(2 * i, 0)
        _gather_chunk(2 * i + 1, 1)

    for b in range(2):
        ob, smo = out_bufs[b]
        pltpu.make_async_copy(ob, out_hbm.at[pl.ds(base, C3)], smo).wait()


_k23 = functools.partial(
    pl.kernel,
    out_type=jax.ShapeDtypeStruct((N_TOKENS,), jnp.float32),
    mesh=_mesh,
    compiler_params=pltpu.CompilerParams(use_tc_tiling_on_sc=False,
                                         needs_layout_passes=False,
                                         disable_bounds_checks=True),
    scratch_types=[
        pltpu.VMEM_SHARED((TP,), jnp.float32),
        pltpu.VMEM((HB, 8), jnp.float32),
        pltpu.VMEM((HB, 8), jnp.float32),
        pltpu.VMEM((HB, 8), jnp.float32),
        pltpu.VMEM((HB, 8), jnp.float32),
        pltpu.VMEM((HB,), jnp.float32),
        pltpu.VMEM((CH,), jnp.float32),
        pltpu.VMEM((TP,), jnp.float32),
        pltpu.VMEM((C3,), jnp.int32),
        pltpu.VMEM((C3,), jnp.int32),
        pltpu.VMEM((C3,), jnp.float32),
        pltpu.VMEM((C3,), jnp.float32),
        pltpu.SemaphoreType.DMA,
        pltpu.SemaphoreType.DMA,
        pltpu.SemaphoreType.DMA,
        pltpu.SemaphoreType.DMA,
        pltpu.SemaphoreType.DMA,
        pltpu.SemaphoreType.DMA,
    ],
)(_k23_body)


def kernel(byte_params, start_pos, token_id, text_len):
    bp = byte_params.astype(jnp.float32)
    aug = jnp.concatenate(
        [bp, jnp.ones((VOCAB, 1), jnp.float32),
         jnp.zeros((VOCAB, 3), jnp.float32)], axis=1)
    zeros_stripe = jnp.zeros((STRIPE, 8), jnp.float32)
    partial = _k1(aug, start_pos, token_id, zeros_stripe)
    return _k23(partial, start_pos)


# speedup vs baseline: 1.1382x; 1.0386x over previous
import functools

import jax
import jax.numpy as jnp
from jax import lax
from jax.experimental import pallas as pl
from jax.experimental.pallas import tpu as pltpu
from jax.experimental.pallas import tpu_sc as plsc

N_TOKENS = 3200000
TEXT_LEN = 100000
TP = 100096
GP = 8
VOCAB = 64
L = 4

NC = 2
NS = 16
NW = NC * NS
TPW = N_TOKENS // NW
STRIPE = TP // NS

C1 = 3200
NCH1 = TPW // C1
CT = TPW - NCH1 * C1
C3 = 2000
NCH3 = TPW // C3
CH = 368
NCHD = STRIPE // CH
NR = CH + 2 * GP
HB = CH + 48
_mesh = plsc.VectorSubcoreMesh(core_axis_name="c", subcore_axis_name="s")


def _k1_body(aug_hbm, sp_hbm, tid_hbm, zeros_hbm, out_hbm,
             aug_sh, s_sh, pos_a, pos_b, pos_c, pos_d, tid_a, tid_b,
             vals_a, vals_b, pos_t, tid_t, vals_t,
             sem_p, sem_t, sem_g, sem_s0, sem_s1, sem_pt, sem_tt, sem_st):
    c = lax.axis_index("c")
    s = lax.axis_index("s")
    wid = s * NC + c
    base = wid * TPW
    r0 = s * STRIPE

    pos_bufs = [pos_a, pos_b, pos_c, pos_d]
    tid_bufs = [tid_a, tid_b]
    vals_bufs = [vals_a, vals_b]

    pend = [(pltpu.async_copy(sp_hbm.at[pl.ds(base, C1)], pos_a, sem_p),
             pltpu.async_copy(tid_hbm.at[pl.ds(base, C1)], tid_a, sem_t))]
    tb = base + NCH1 * C1
    tail_in = (pltpu.async_copy(sp_hbm.at[pl.ds(tb, CT)], pos_t, sem_pt),
               pltpu.async_copy(tid_hbm.at[pl.ds(tb, CT)], tid_t, sem_tt))

    pltpu.sync_copy(zeros_hbm, s_sh.at[pl.ds(r0, STRIPE)])

    @pl.when(s == 0)
    def _():
        pltpu.sync_copy(aug_hbm, aug_sh)
        pltpu.sync_copy(zeros_hbm.at[pl.ds(0, GP)], out_hbm.at[c, pl.ds(0, GP)])
        pltpu.sync_copy(zeros_hbm.at[pl.ds(0, GP)],
                        out_hbm.at[c, pl.ds(GP + TP, GP)])

    plsc.subcore_barrier()

    sc_pend = []
    for j in range(NCH1):
        cpj, ctj = pend.pop()
        cpj.wait()
        ctj.wait()
        if j + 1 < NCH1:
            nb = base + (j + 1) * C1
            pend.append(
                (pltpu.async_copy(sp_hbm.at[pl.ds(nb, C1)],
                                  pos_bufs[(j + 1) % 4], sem_p),
                 pltpu.async_copy(tid_hbm.at[pl.ds(nb, C1)],
                                  tid_bufs[(j + 1) % 2], sem_t)))
        if len(sc_pend) >= 2:
            sc_pend.pop(0).wait()
        vb = vals_bufs[j % 2]
        pltpu.async_copy(aug_sh.at[tid_bufs[j % 2]], vb, sem_g).wait()
        sc_pend.append(
            pltpu.async_copy(vb, s_sh.at[pos_bufs[j % 4]],
                             [sem_s0, sem_s1][j % 2], add=True))

    tail_in[0].wait()
    tail_in[1].wait()
    pltpu.async_copy(aug_sh.at[tid_t], vals_t, sem_g).wait()
    sc_pend.append(
        pltpu.async_copy(vals_t, s_sh.at[pos_t], sem_st, add=True))

    for d in sc_pend:
        d.wait()

    plsc.subcore_barrier()
    pltpu.sync_copy(s_sh.at[pl.ds(r0, STRIPE)],
                    out_hbm.at[c, pl.ds(GP + r0, STRIPE)])


_k1 = functools.partial(
    pl.kernel,
    out_type=jax.ShapeDtypeStruct((NC, TP + 2 * GP, 8), jnp.float32),
    mesh=_mesh,
    compiler_params=pltpu.CompilerParams(use_tc_tiling_on_sc=False,
                                         disable_bounds_checks=True),
    scratch_types=[
        pltpu.VMEM_SHARED((VOCAB, 8), jnp.float32),
        pltpu.VMEM_SHARED((TP, 8), jnp.float32),
        pltpu.VMEM((C1,), jnp.int32),
        pltpu.VMEM((C1,), jnp.int32),
        pltpu.VMEM((C1,), jnp.int32),
        pltpu.VMEM((C1,), jnp.int32),
        pltpu.VMEM((C1,), jnp.int32),
        pltpu.VMEM((C1,), jnp.int32),
        pltpu.VMEM((C1, 8), jnp.float32),
        pltpu.VMEM((C1, 8), jnp.float32),
        pltpu.VMEM((CT,), jnp.int32),
        pltpu.VMEM((CT,), jnp.int32),
        pltpu.VMEM((CT, 8), jnp.float32),
        pltpu.SemaphoreType.DMA,
        pltpu.SemaphoreType.DMA,
        pltpu.SemaphoreType.DMA,
        pltpu.SemaphoreType.DMA,
        pltpu.SemaphoreType.DMA,
        pltpu.SemaphoreType.DMA,
        pltpu.SemaphoreType.DMA,
        pltpu.SemaphoreType.DMA,
    ],
)(_k1_body)


def _k23_body(part_hbm, sp_hbm, out_hbm,
              w4_sh, h0a, h1a, h0b, h1b, bw_v, w4c_v, w4_v,
              idx_a, idx_b, out_a, out_b,
              sem_ha, sem_hb, sem_ia, sem_ib, sem_oa, sem_ob):
    c = lax.axis_index("c")
    s = lax.axis_index("s")
    wid = s * NC + c
    base = wid * TPW
    r0 = s * STRIPE

    iota = lax.iota(jnp.int32, 16)
    c4 = jnp.full((16,), 4, jnp.int32)

    hb_bufs = [(h0a, h1a, sem_ha), (h0b, h1b, sem_hb)]

    def _halo_issue(j, b):
        a = r0 + j * CH
        b0, b1, sm = hb_bufs[b]
        pltpu.async_copy(part_hbm.at[0, pl.ds(a, NR)], b0.at[pl.ds(0, NR)], sm)
        pltpu.async_copy(part_hbm.at[1, pl.ds(a, NR)], b1.at[pl.ds(0, NR)], sm)

    def _halo_wait(b):
        b0, b1, sm = hb_bufs[b]
        pltpu.make_async_copy(part_hbm.at[0, pl.ds(0, NR)],
                              b0.at[pl.ds(0, NR)], sm).wait()
        pltpu.make_async_copy(part_hbm.at[1, pl.ds(0, NR)],
                              b1.at[pl.ds(0, NR)], sm).wait()

    def _dense_chunk(j, b):
        a = r0 + j * CH
        _halo_wait(b)
        h0, h1, _ = hb_bufs[b]

        ng1 = (CH + 3 + 15) // 16

        @plsc.parallel_loop(0, ng1, unroll=4)
        def _(g):
            l = pl.multiple_of(g * 16, 16) + GP
            rq = jnp.full((16,), l, jnp.int32) + iota
            bsum = jnp.zeros((16,), jnp.float32)
            bcnt = jnp.zeros((16,), jnp.float32)
            for k in range(4):
                rk = rq - k
                ck = jnp.full((16,), k, jnp.int32)
                bsum = bsum + plsc.load_gather(h0, [rk, ck])
                bsum = bsum + plsc.load_gather(h1, [rk, ck])
                bcnt = bcnt + plsc.load_gather(h0, [rk, c4])
                bcnt = bcnt + plsc.load_gather(h1, [rk, c4])
            bw = jnp.where(bcnt > 0, bsum / jnp.maximum(bcnt, 1.0), 0.0)
            bw_v[pl.ds(l, 16)] = bw

        @plsc.parallel_loop(0, CH // 16, unroll=4)
        def _(g):
            o = pl.multiple_of(g * 16, 16)
            rq = jnp.full((16,), o + GP, jnp.int32) + iota
            w = bw_v[pl.ds(o + GP, 16)]
            for k in range(1, 4):
                w = w + plsc.load_gather(bw_v, [rq + k])
            w4c_v[pl.ds(o, 16)] = w

        pltpu.sync_copy(w4c_v, w4_sh.at[pl.ds(a, CH)])

        @pl.when(j + 2 < NCHD)
        def _():
            _halo_issue(j + 2, b)

    _halo_issue(0, 0)
    _halo_issue(1, 1)

    @pl.loop(0, NCHD // 2)
    def _(i):
        _dense_chunk(2 * i, 0)
        _dense_chunk(2 * i + 1, 1)

    _dense_chunk(NCHD - 1, 0)

    plsc.subcore_barrier()

    pltpu.sync_copy(w4_sh, w4_v)

    idx_bufs = [(idx_a, sem_ia), (idx_b, sem_ib)]
    out_bufs = [(out_a, sem_oa), (out_b, sem_ob)]

    def _idx_issue(j, b):
        ib, sm = idx_bufs[b]
        pltpu.async_copy(sp_hbm.at[pl.ds(base + j * C3, C3)], ib, sm)

    def _gather_chunk(j, b):
        ib, smi = idx_bufs[b]
        ob, smo = out_bufs[b]
        pltpu.make_async_copy(sp_hbm.at[pl.ds(base, C3)], ib, smi).wait()

        @pl.when(j >= 2)
        def _():
            pltpu.make_async_copy(ob, out_hbm.at[pl.ds(base, C3)], smo).wait()

        @plsc.parallel_loop(0, C3 // 16, unroll=16)
        def _(g):
            off = pl.multiple_of(g * 16, 16)
            idx = ib[pl.ds(off, 16)]
            ob[pl.ds(off, 16)] = plsc.load_gather(w4_v, [idx])

        pltpu.async_copy(ob, out_hbm.at[pl.ds(base + j * C3, C3)], smo)

        @pl.when(j + 2 < NCH3)
        def _():
            _idx_issue(j + 2, b)

    _idx_issue(0, 0)
    _idx_issue(1, 1)

    @pl.loop(0, NCH3 // 2)
    def _(i):
        _gather_chunk(2 * i, 0)
        _gather_chunk(2 * i + 1, 1)

    for b in range(2):
        ob, smo = out_bufs[b]
        pltpu.make_async_copy(ob, out_hbm.at[pl.ds(base, C3)], smo).wait()


_k23 = functools.partial(
    pl.kernel,
    out_type=jax.ShapeDtypeStruct((N_TOKENS,), jnp.float32),
    mesh=_mesh,
    compiler_params=pltpu.CompilerParams(use_tc_tiling_on_sc=False,
                                         needs_layout_passes=False,
                                         disable_bounds_checks=True),
    scratch_types=[
        pltpu.VMEM_SHARED((TP,), jnp.float32),
        pltpu.VMEM((HB, 8), jnp.float32),
        pltpu.VMEM((HB, 8), jnp.float32),
        pltpu.VMEM((HB, 8), jnp.float32),
        pltpu.VMEM((HB, 8), jnp.float32),
        pltpu.VMEM((HB,), jnp.float32),
        pltpu.VMEM((CH,), jnp.float32),
        pltpu.VMEM((TP,), jnp.float32),
        pltpu.VMEM((C3,), jnp.int32),
        pltpu.VMEM((C3,), jnp.int32),
        pltpu.VMEM((C3,), jnp.float32),
        pltpu.VMEM((C3,), jnp.float32),
        pltpu.SemaphoreType.DMA,
        pltpu.SemaphoreType.DMA,
        pltpu.SemaphoreType.DMA,
        pltpu.SemaphoreType.DMA,
        pltpu.SemaphoreType.DMA,
        pltpu.SemaphoreType.DMA,
    ],
)(_k23_body)


def kernel(byte_params, start_pos, token_id, text_len):
    bp = byte_params.astype(jnp.float32)
    aug = jnp.concatenate(
        [bp, jnp.ones((VOCAB, 1), jnp.float32),
         jnp.zeros((VOCAB, 3), jnp.float32)], axis=1)
    zeros_stripe = jnp.zeros((STRIPE, 8), jnp.float32)
    partial = _k1(aug, start_pos, token_id, zeros_stripe)
    return _k23(partial, start_pos)


# speedup vs baseline: 1.1421x; 1.0035x over previous
import functools

import jax
import jax.numpy as jnp
from jax import lax
from jax.experimental import pallas as pl
from jax.experimental.pallas import tpu as pltpu
from jax.experimental.pallas import tpu_sc as plsc

N_TOKENS = 3200000
TEXT_LEN = 100000
TP = 100096
GP = 8
VOCAB = 64
L = 4

NC = 2
NS = 16
NW = NC * NS
TPW = N_TOKENS // NW
STRIPE = TP // NS

C1 = 3200
NCH1 = TPW // C1
CT = TPW - NCH1 * C1
C3 = 2000
NCH3 = TPW // C3
CH = 368
NCHD = STRIPE // CH
NR = CH + 2 * GP
HB = CH + 48
_mesh = plsc.VectorSubcoreMesh(core_axis_name="c", subcore_axis_name="s")


def _k1_body(aug_hbm, sp_hbm, tid_hbm, zeros_hbm, out_hbm,
             aug_sh, s_sh, pos_a, pos_b, pos_c, pos_d, tid_a, tid_b,
             vals_a, vals_b, pos_t, tid_t, vals_t,
             sem_p, sem_t, sem_g, sem_s0, sem_s1, sem_pt, sem_tt, sem_st):
    c = lax.axis_index("c")
    s = lax.axis_index("s")
    wid = s * NC + c
    base = wid * TPW
    r0 = s * STRIPE

    pos_bufs = [pos_a, pos_b, pos_c, pos_d]
    tid_bufs = [tid_a, tid_b]
    vals_bufs = [vals_a, vals_b]

    pend = [(pltpu.async_copy(sp_hbm.at[pl.ds(base, C1)], pos_a, sem_p),
             pltpu.async_copy(tid_hbm.at[pl.ds(base, C1)], tid_a, sem_t))]
    tb = base + NCH1 * C1
    tail_in = (pltpu.async_copy(sp_hbm.at[pl.ds(tb, CT)], pos_t, sem_pt),
               pltpu.async_copy(tid_hbm.at[pl.ds(tb, CT)], tid_t, sem_tt))

    pltpu.sync_copy(zeros_hbm, s_sh.at[pl.ds(r0, STRIPE)])

    @pl.when(s == 0)
    def _():
        pltpu.sync_copy(aug_hbm, aug_sh)
        pltpu.sync_copy(zeros_hbm.at[pl.ds(0, GP)], out_hbm.at[c, pl.ds(0, GP)])
        pltpu.sync_copy(zeros_hbm.at[pl.ds(0, GP)],
                        out_hbm.at[c, pl.ds(GP + TP, GP)])

    plsc.subcore_barrier()

    sc_pend = []
    for j in range(NCH1):
        cpj, ctj = pend.pop()
        cpj.wait()
        ctj.wait()
        if j + 1 < NCH1:
            nb = base + (j + 1) * C1
            pend.append(
                (pltpu.async_copy(sp_hbm.at[pl.ds(nb, C1)],
                                  pos_bufs[(j + 1) % 4], sem_p),
                 pltpu.async_copy(tid_hbm.at[pl.ds(nb, C1)],
                                  tid_bufs[(j + 1) % 2], sem_t)))
        if len(sc_pend) >= 2:
            sc_pend.pop(0).wait()
        vb = vals_bufs[j % 2]
        pltpu.async_copy(aug_sh.at[tid_bufs[j % 2]], vb, sem_g).wait()
        sc_pend.append(
            pltpu.async_copy(vb, s_sh.at[pos_bufs[j % 4]],
                             [sem_s0, sem_s1][j % 2], add=True))

    tail_in[0].wait()
    tail_in[1].wait()
    pltpu.async_copy(aug_sh.at[tid_t], vals_t, sem_g).wait()
    sc_pend.append(
        pltpu.async_copy(vals_t, s_sh.at[pos_t], sem_st, add=True))

    for d in sc_pend:
        d.wait()

    plsc.subcore_barrier()
    pltpu.sync_copy(s_sh.at[pl.ds(r0, STRIPE)],
                    out_hbm.at[c, pl.ds(GP + r0, STRIPE)])


_k1 = functools.partial(
    pl.kernel,
    out_type=jax.ShapeDtypeStruct((NC, TP + 2 * GP, 8), jnp.float32),
    mesh=_mesh,
    compiler_params=pltpu.CompilerParams(use_tc_tiling_on_sc=False,
                                         disable_bounds_checks=True),
    scratch_types=[
        pltpu.VMEM_SHARED((VOCAB, 8), jnp.float32),
        pltpu.VMEM_SHARED((TP, 8), jnp.float32),
        pltpu.VMEM((C1,), jnp.int32),
        pltpu.VMEM((C1,), jnp.int32),
        pltpu.VMEM((C1,), jnp.int32),
        pltpu.VMEM((C1,), jnp.int32),
        pltpu.VMEM((C1,), jnp.int32),
        pltpu.VMEM((C1,), jnp.int32),
        pltpu.VMEM((C1, 8), jnp.float32),
        pltpu.VMEM((C1, 8), jnp.float32),
        pltpu.VMEM((CT,), jnp.int32),
        pltpu.VMEM((CT,), jnp.int32),
        pltpu.VMEM((CT, 8), jnp.float32),
        pltpu.SemaphoreType.DMA,
        pltpu.SemaphoreType.DMA,
        pltpu.SemaphoreType.DMA,
        pltpu.SemaphoreType.DMA,
        pltpu.SemaphoreType.DMA,
        pltpu.SemaphoreType.DMA,
        pltpu.SemaphoreType.DMA,
        pltpu.SemaphoreType.DMA,
    ],
)(_k1_body)


def _k23_body(part_hbm, sp_hbm, out_hbm,
              w4_sh, h0a, h1a, h0b, h1b, bw_v, w4c_v, w4_v,
              idx_a, idx_b, out_a, out_b,
              sem_ha, sem_hb, sem_ia, sem_ib, sem_oa, sem_ob):
    c = lax.axis_index("c")
    s = lax.axis_index("s")
    wid = s * NC + c
    base = wid * TPW
    r0 = s * STRIPE

    iota = lax.iota(jnp.int32, 16)
    c4 = jnp.full((16,), 4, jnp.int32)

    hb_bufs = [(h0a, h1a, sem_ha), (h0b, h1b, sem_hb)]

    def _halo_issue(j, b):
        a = r0 + j * CH
        b0, b1, sm = hb_bufs[b]
        pltpu.async_copy(part_hbm.at[0, pl.ds(a, NR)], b0.at[pl.ds(0, NR)], sm)
        pltpu.async_copy(part_hbm.at[1, pl.ds(a, NR)], b1.at[pl.ds(0, NR)], sm)

    def _halo_wait(b):
        b0, b1, sm = hb_bufs[b]
        pltpu.make_async_copy(part_hbm.at[0, pl.ds(0, NR)],
                              b0.at[pl.ds(0, NR)], sm).wait()
        pltpu.make_async_copy(part_hbm.at[1, pl.ds(0, NR)],
                              b1.at[pl.ds(0, NR)], sm).wait()

    def _dense_chunk(j, b):
        a = r0 + j * CH
        _halo_wait(b)
        h0, h1, _ = hb_bufs[b]

        ng1 = (CH + 3 + 15) // 16

        @plsc.parallel_loop(0, ng1, unroll=4)
        def _(g):
            l = pl.multiple_of(g * 16, 16) + GP
            rq = jnp.full((16,), l, jnp.int32) + iota
            bsum = jnp.zeros((16,), jnp.float32)
            bcnt = jnp.zeros((16,), jnp.float32)
            for k in range(4):
                rk = rq - k
                ck = jnp.full((16,), k, jnp.int32)
                bsum = bsum + plsc.load_gather(h0, [rk, ck])
                bsum = bsum + plsc.load_gather(h1, [rk, ck])
                bcnt = bcnt + plsc.load_gather(h0, [rk, c4])
                bcnt = bcnt + plsc.load_gather(h1, [rk, c4])
            bw = jnp.where(bcnt > 0, bsum / jnp.maximum(bcnt, 1.0), 0.0)
            bw_v[pl.ds(l, 16)] = bw

        @plsc.parallel_loop(0, CH // 16, unroll=4)
        def _(g):
            o = pl.multiple_of(g * 16, 16)
            rq = jnp.full((16,), o + GP, jnp.int32) + iota
            w = bw_v[pl.ds(o + GP, 16)]
            for k in range(1, 4):
                w = w + plsc.load_gather(bw_v, [rq + k])
            w4c_v[pl.ds(o, 16)] = w

        pltpu.sync_copy(w4c_v, w4_sh.at[pl.ds(a, CH)])

        @pl.when(j + 2 < NCHD)
        def _():
            _halo_issue(j + 2, b)

    idx_bufs = [(idx_a, sem_ia), (idx_b, sem_ib)]
    out_bufs = [(out_a, sem_oa), (out_b, sem_ob)]

    def _idx_issue(j, b):
        ib, sm = idx_bufs[b]
        pltpu.async_copy(sp_hbm.at[pl.ds(base + j * C3, C3)], ib, sm)

    _halo_issue(0, 0)
    _halo_issue(1, 1)
    _idx_issue(0, 0)
    _idx_issue(1, 1)

    @pl.loop(0, NCHD // 2)
    def _(i):
        _dense_chunk(2 * i, 0)
        _dense_chunk(2 * i + 1, 1)

    _dense_chunk(NCHD - 1, 0)

    plsc.subcore_barrier()

    pltpu.sync_copy(w4_sh, w4_v)

    def _gather_chunk(j, b):
        ib, smi = idx_bufs[b]
        ob, smo = out_bufs[b]
        pltpu.make_async_copy(sp_hbm.at[pl.ds(base, C3)], ib, smi).wait()

        @pl.when(j >= 2)
        def _():
            pltpu.make_async_copy(ob, out_hbm.at[pl.ds(base, C3)], smo).wait()

        @plsc.parallel_loop(0, C3 // 16, unroll=16)
        def _(g):
            off = pl.multiple_of(g * 16, 16)
            idx = ib[pl.ds(off, 16)]
            ob[pl.ds(off, 16)] = plsc.load_gather(w4_v, [idx])

        pltpu.async_copy(ob, out_hbm.at[pl.ds(base + j * C3, C3)], smo)

        @pl.when(j + 2 < NCH3)
        def _():
            _idx_issue(j + 2, b)

    @pl.loop(0, NCH3 // 2)
    def _(i):
        _gather_chunk(2 * i, 0)
        _gather_chunk(2 * i + 1, 1)

    for b in range(2):
        ob, smo = out_bufs[b]
        pltpu.make_async_copy(ob, out_hbm.at[pl.ds(base, C3)], smo).wait()


_k23 = functools.partial(
    pl.kernel,
    out_type=jax.ShapeDtypeStruct((N_TOKENS,), jnp.float32),
    mesh=_mesh,
    compiler_params=pltpu.CompilerParams(use_tc_tiling_on_sc=False,
                                         needs_layout_passes=False,
                                         disable_bounds_checks=True),
    scratch_types=[
        pltpu.VMEM_SHARED((TP,), jnp.float32),
        pltpu.VMEM((HB, 8), jnp.float32),
        pltpu.VMEM((HB, 8), jnp.float32),
        pltpu.VMEM((HB, 8), jnp.float32),
        pltpu.VMEM((HB, 8), jnp.float32),
        pltpu.VMEM((HB,), jnp.float32),
        pltpu.VMEM((CH,), jnp.float32),
        pltpu.VMEM((TP,), jnp.float32),
        pltpu.VMEM((C3,), jnp.int32),
        pltpu.VMEM((C3,), jnp.int32),
        pltpu.VMEM((C3,), jnp.float32),
        pltpu.VMEM((C3,), jnp.float32),
        pltpu.SemaphoreType.DMA,
        pltpu.SemaphoreType.DMA,
        pltpu.SemaphoreType.DMA,
        pltpu.SemaphoreType.DMA,
        pltpu.SemaphoreType.DMA,
        pltpu.SemaphoreType.DMA,
    ],
)(_k23_body)


def kernel(byte_params, start_pos, token_id, text_len):
    bp = byte_params.astype(jnp.float32)
    aug = jnp.concatenate(
        [bp, jnp.ones((VOCAB, 1), jnp.float32),
         jnp.zeros((VOCAB, 3), jnp.float32)], axis=1)
    zeros_stripe = jnp.zeros((STRIPE, 8), jnp.float32)
    partial = _k1(aug, start_pos, token_id, zeros_stripe)
    return _k23(partial, start_pos)
